# baseline (speedup 1.0000x reference)
import functools

import jax
import jax.numpy as jnp
import numpy as np
from jax import lax
from jax.experimental import pallas as pl
from jax.experimental.pallas import tpu as pltpu
from jax.experimental.pallas import tpu_sc as plsc

_INDEX_NP = np.array(
    [0, 7777, 15554, 23331, 31108, 38885, 46662, 54439, 62216, 69993,
     77770, 85547, 93324, 101101, 108878, 116655, 124432, 132209, 139986,
     147763, 155540, 163317, 171094, 178871, 186648, 194425, 202202,
     209979, 217756, 225533, 233310, 241087, 248864, 256641, 264418,
     272195, 279972, 287749, 295526, 303303, 311080, 318857, 326634,
     334411, 342188, 349965, 357742, 365519, 373296, 381073, 388850,
     396627, 404404, 412181, 419958, 427735, 435512, 443289, 451066,
     458843, 466620, 474397, 482174, 489951], dtype=np.int32)

_B = 64
_D = 64
_V = 1_000_000
_RPW = 8
_NACT = _B // _RPW
_L = 16

_SUPER = [int(v) // 2 for v in _INDEX_NP]
_HOFF = [(int(v) % 2) * _D for v in _INDEX_NP]

_mesh = plsc.VectorSubcoreMesh(core_axis_name="c", subcore_axis_name="s")


@functools.partial(
    pl.kernel,
    mesh=_mesh,
    out_type=jax.ShapeDtypeStruct((_B, _D), jnp.float32),
    scratch_types=[
        pltpu.VMEM((_RPW, 2 * _D), jnp.float32),
        pltpu.VMEM((_RPW, _D), jnp.float32),
        pltpu.SemaphoreType.DMA,
    ],
)
def _gather_rows(table_hbm, out_hbm, rows_v, out_v, sem):
    wid = lax.axis_index("s") * 2 + lax.axis_index("c")

    for k in range(_NACT):
        @pl.when(wid == k)
        def _(k=k):
            copies = [
                pltpu.async_copy(
                    table_hbm.at[pl.ds(_SUPER[k * _RPW + j], 1)],
                    rows_v.at[pl.ds(j, 1)],
                    sem,
                )
                for j in range(_RPW)
            ]
            for c in copies:
                c.wait()
            for j in range(_RPW):
                off = _HOFF[k * _RPW + j]
                for c in range(_D // _L):
                    out_v[j, pl.ds(c * _L, _L)] = rows_v[j, pl.ds(off + c * _L, _L)]
            pltpu.sync_copy(out_v, out_hbm.at[pl.ds(k * _RPW, _RPW)])


def kernel(x):
    return _gather_rows(x.reshape(_V // 2, 2 * _D))


# speedup vs baseline: 24.4487x; 24.4487x over previous
import functools

import jax
import jax.numpy as jnp
import numpy as np
from jax import lax
from jax.experimental import pallas as pl
from jax.experimental.pallas import tpu as pltpu
from jax.experimental.pallas import tpu_sc as plsc

_INDEX_NP = np.array(
    [0, 7777, 15554, 23331, 31108, 38885, 46662, 54439, 62216, 69993,
     77770, 85547, 93324, 101101, 108878, 116655, 124432, 132209, 139986,
     147763, 155540, 163317, 171094, 178871, 186648, 194425, 202202,
     209979, 217756, 225533, 233310, 241087, 248864, 256641, 264418,
     272195, 279972, 287749, 295526, 303303, 311080, 318857, 326634,
     334411, 342188, 349965, 357742, 365519, 373296, 381073, 388850,
     396627, 404404, 412181, 419958, 427735, 435512, 443289, 451066,
     458843, 466620, 474397, 482174, 489951], dtype=np.int32)

_B = 64
_D = 64
_LANES = 128
_CPW = 8
_NACT = _B // _CPW
_L = 16

_BASE = [(int(v) // _LANES) * _LANES for v in _INDEX_NP]
_LANE = [int(v) % _LANES for v in _INDEX_NP]

_mesh = plsc.VectorSubcoreMesh(core_axis_name="c", subcore_axis_name="s")


@functools.partial(
    pl.kernel,
    mesh=_mesh,
    out_type=jax.ShapeDtypeStruct((_B, _D), jnp.float32),
    scratch_types=[
        pltpu.VMEM((_CPW, _D, _LANES), jnp.float32),
        pltpu.VMEM((_CPW, _D), jnp.float32),
        pltpu.SemaphoreType.DMA,
    ],
    compiler_params=pltpu.CompilerParams(needs_layout_passes=False),
)
def _gather_rows(table_t_hbm, out_hbm, blk_v, out_v, sem):
    wid = lax.axis_index("s") * 2 + lax.axis_index("c")

    for k in range(_NACT):
        @pl.when(wid == k)
        def _(k=k):
            copies = [
                pltpu.async_copy(
                    table_t_hbm.at[:, pl.ds(_BASE[k * _CPW + j], _LANES)],
                    blk_v.at[j],
                    sem,
                )
                for j in range(_CPW)
            ]
            for c in copies:
                c.wait()
            seq = lax.iota(jnp.int32, _L)
            for j in range(_CPW):
                lane = jnp.full((_L,), _LANE[k * _CPW + j], jnp.int32)
                blk = jnp.full((_L,), j, jnp.int32)
                for q in range(_D // _L):
                    vals = plsc.load_gather(blk_v, [blk, seq + q * _L, lane])
                    out_v[j, pl.ds(q * _L, _L)] = vals
            pltpu.sync_copy(out_v, out_hbm.at[pl.ds(k * _CPW, _CPW)])


def kernel(x):
    return _gather_rows(x.T)


# speedup vs baseline: 26.0033x; 1.0636x over previous
import functools

import jax
import jax.numpy as jnp
import numpy as np
from jax import lax
from jax.experimental import pallas as pl
from jax.experimental.pallas import tpu as pltpu
from jax.experimental.pallas import tpu_sc as plsc

_INDEX_NP = np.array(
    [0, 7777, 15554, 23331, 31108, 38885, 46662, 54439, 62216, 69993,
     77770, 85547, 93324, 101101, 108878, 116655, 124432, 132209, 139986,
     147763, 155540, 163317, 171094, 178871, 186648, 194425, 202202,
     209979, 217756, 225533, 233310, 241087, 248864, 256641, 264418,
     272195, 279972, 287749, 295526, 303303, 311080, 318857, 326634,
     334411, 342188, 349965, 357742, 365519, 373296, 381073, 388850,
     396627, 404404, 412181, 419958, 427735, 435512, 443289, 451066,
     458843, 466620, 474397, 482174, 489951], dtype=np.int32)

_B = 64
_D = 64
_LANES = 128
_CPW = 2
_NACT = _B // _CPW
_L = 16

_BASE = [(int(v) // _LANES) * _LANES for v in _INDEX_NP]
_LANE = [int(v) % _LANES for v in _INDEX_NP]

_mesh = plsc.VectorSubcoreMesh(core_axis_name="c", subcore_axis_name="s")


@functools.partial(
    pl.kernel,
    mesh=_mesh,
    out_type=jax.ShapeDtypeStruct((_B, _D), jnp.float32),
    scratch_types=[
        pltpu.VMEM((_CPW, _D, _LANES), jnp.float32),
        pltpu.VMEM((_CPW, _D), jnp.float32),
        pltpu.SemaphoreType.DMA,
    ],
    compiler_params=pltpu.CompilerParams(needs_layout_passes=False),
)
def _gather_rows(table_t_hbm, out_hbm, blk_v, out_v, sem):
    wid = lax.axis_index("s") * 2 + lax.axis_index("c")

    for k in range(_NACT):
        @pl.when(wid == k)
        def _(k=k):
            copies = [
                pltpu.async_copy(
                    table_t_hbm.at[:, pl.ds(_BASE[k * _CPW + j], _LANES)],
                    blk_v.at[j],
                    sem,
                )
                for j in range(_CPW)
            ]
            for c in copies:
                c.wait()
            seq = lax.iota(jnp.int32, _L)
            for j in range(_CPW):
                lane = jnp.full((_L,), _LANE[k * _CPW + j], jnp.int32)
                blk = jnp.full((_L,), j, jnp.int32)
                for q in range(_D // _L):
                    vals = plsc.load_gather(blk_v, [blk, seq + q * _L, lane])
                    out_v[j, pl.ds(q * _L, _L)] = vals
            pltpu.sync_copy(out_v, out_hbm.at[pl.ds(k * _CPW, _CPW)])


def kernel(x):
    return _gather_rows(x.T)


# speedup vs baseline: 27.2793x; 1.0491x over previous
import functools

import jax
import jax.numpy as jnp
import numpy as np
from jax import lax
from jax.experimental import pallas as pl
from jax.experimental.pallas import tpu as pltpu
from jax.experimental.pallas import tpu_sc as plsc

_INDEX_NP = np.array(
    [0, 7777, 15554, 23331, 31108, 38885, 46662, 54439, 62216, 69993,
     77770, 85547, 93324, 101101, 108878, 116655, 124432, 132209, 139986,
     147763, 155540, 163317, 171094, 178871, 186648, 194425, 202202,
     209979, 217756, 225533, 233310, 241087, 248864, 256641, 264418,
     272195, 279972, 287749, 295526, 303303, 311080, 318857, 326634,
     334411, 342188, 349965, 357742, 365519, 373296, 381073, 388850,
     396627, 404404, 412181, 419958, 427735, 435512, 443289, 451066,
     458843, 466620, 474397, 482174, 489951], dtype=np.int32)

_B = 64
_D = 64
_LANES = 128
_CPW = 4
_NACT = _B // _CPW
_L = 16

_BASE = [(int(v) // _LANES) * _LANES for v in _INDEX_NP]
_LANE = [int(v) % _LANES for v in _INDEX_NP]

_mesh = plsc.VectorSubcoreMesh(core_axis_name="c", subcore_axis_name="s", num_cores=1)


@functools.partial(
    pl.kernel,
    mesh=_mesh,
    out_type=jax.ShapeDtypeStruct((_B, _D), jnp.float32),
    scratch_types=[
        pltpu.VMEM((_CPW, _D, _LANES), jnp.float32),
        pltpu.VMEM((_CPW, _D), jnp.float32),
        pltpu.SemaphoreType.DMA,
    ],
    compiler_params=pltpu.CompilerParams(needs_layout_passes=False),
)
def _gather_rows(table_t_hbm, out_hbm, blk_v, out_v, sem):
    wid = lax.axis_index("s")

    for k in range(_NACT):
        @pl.when(wid == k)
        def _(k=k):
            copies = [
                pltpu.async_copy(
                    table_t_hbm.at[:, pl.ds(_BASE[k * _CPW + j], _LANES)],
                    blk_v.at[j],
                    sem,
                )
                for j in range(_CPW)
            ]
            for c in copies:
                c.wait()
            seq = lax.iota(jnp.int32, _L)
            for j in range(_CPW):
                lane = jnp.full((_L,), _LANE[k * _CPW + j], jnp.int32)
                blk = jnp.full((_L,), j, jnp.int32)
                for q in range(_D // _L):
                    vals = plsc.load_gather(blk_v, [blk, seq + q * _L, lane])
                    out_v[j, pl.ds(q * _L, _L)] = vals
            pltpu.sync_copy(out_v, out_hbm.at[pl.ds(k * _CPW, _CPW)])


def kernel(x):
    return _gather_rows(x.T)


# speedup vs baseline: 27.4391x; 1.0059x over previous
import functools

import jax
import jax.numpy as jnp
import numpy as np
from jax import lax
from jax.experimental import pallas as pl
from jax.experimental.pallas import tpu as pltpu
from jax.experimental.pallas import tpu_sc as plsc

_INDEX_NP = np.array(
    [0, 7777, 15554, 23331, 31108, 38885, 46662, 54439, 62216, 69993,
     77770, 85547, 93324, 101101, 108878, 116655, 124432, 132209, 139986,
     147763, 155540, 163317, 171094, 178871, 186648, 194425, 202202,
     209979, 217756, 225533, 233310, 241087, 248864, 256641, 264418,
     272195, 279972, 287749, 295526, 303303, 311080, 318857, 326634,
     334411, 342188, 349965, 357742, 365519, 373296, 381073, 388850,
     396627, 404404, 412181, 419958, 427735, 435512, 443289, 451066,
     458843, 466620, 474397, 482174, 489951], dtype=np.int32)

_B = 64
_D = 64
_LANES = 128
_CPW = 4
_NACT = _B // _CPW
_L = 16

_BASE = [(int(v) // _LANES) * _LANES for v in _INDEX_NP]
_LANE = [int(v) % _LANES for v in _INDEX_NP]

_mesh = plsc.VectorSubcoreMesh(core_axis_name="c", subcore_axis_name="s", num_cores=1)


@functools.partial(
    pl.kernel,
    mesh=_mesh,
    out_type=jax.ShapeDtypeStruct((_B, _D), jnp.float32),
    scratch_types=[
        pltpu.VMEM((_CPW, _D, _LANES), jnp.float32),
        pltpu.VMEM((_CPW, _D), jnp.float32),
        pltpu.SemaphoreType.DMA,
    ],
    compiler_params=pltpu.CompilerParams(
        needs_layout_passes=False,
        skip_device_barrier=True,
        disable_bounds_checks=True,
    ),
)
def _gather_rows(table_t_hbm, out_hbm, blk_v, out_v, sem):
    wid = lax.axis_index("s")

    for k in range(_NACT):
        @pl.when(wid == k)
        def _(k=k):
            copies = [
                pltpu.async_copy(
                    table_t_hbm.at[:, pl.ds(_BASE[k * _CPW + j], _LANES)],
                    blk_v.at[j],
                    sem,
                )
                for j in range(_CPW)
            ]
            for c in copies:
                c.wait()
            seq = lax.iota(jnp.int32, _L)
            for j in range(_CPW):
                lane = jnp.full((_L,), _LANE[k * _CPW + j], jnp.int32)
                blk = jnp.full((_L,), j, jnp.int32)
                for q in range(_D // _L):
                    vals = plsc.load_gather(blk_v, [blk, seq + q * _L, lane])
                    out_v[j, pl.ds(q * _L, _L)] = vals
            pltpu.sync_copy(out_v, out_hbm.at[pl.ds(k * _CPW, _CPW)])


def kernel(x):
    return _gather_rows(x.T)


# speedup vs baseline: 30.8994x; 1.1261x over previous
import functools

import jax
import jax.numpy as jnp
import numpy as np
from jax import lax
from jax.experimental import pallas as pl
from jax.experimental.pallas import tpu as pltpu
from jax.experimental.pallas import tpu_sc as plsc

_INDEX_NP = np.array(
    [0, 7777, 15554, 23331, 31108, 38885, 46662, 54439, 62216, 69993,
     77770, 85547, 93324, 101101, 108878, 116655, 124432, 132209, 139986,
     147763, 155540, 163317, 171094, 178871, 186648, 194425, 202202,
     209979, 217756, 225533, 233310, 241087, 248864, 256641, 264418,
     272195, 279972, 287749, 295526, 303303, 311080, 318857, 326634,
     334411, 342188, 349965, 357742, 365519, 373296, 381073, 388850,
     396627, 404404, 412181, 419958, 427735, 435512, 443289, 451066,
     458843, 466620, 474397, 482174, 489951], dtype=np.int32)
_STRIDE = 7777
assert (_INDEX_NP == _STRIDE * np.arange(64, dtype=np.int64)).all()

_B = 64
_D = 64
_LANES = 128
_CPW = 4
_L = 16

_mesh = plsc.VectorSubcoreMesh(core_axis_name="c", subcore_axis_name="s", num_cores=1)


@functools.partial(
    pl.kernel,
    mesh=_mesh,
    out_type=jax.ShapeDtypeStruct((_B, _D), jnp.float32),
    scratch_types=[
        pltpu.VMEM((_CPW, _D, _LANES), jnp.float32),
        pltpu.VMEM((_CPW, _D), jnp.float32),
        pltpu.SemaphoreType.DMA,
    ],
    compiler_params=pltpu.CompilerParams(needs_layout_passes=False),
)
def _gather_rows(table_t_hbm, out_hbm, blk_v, out_v, sem):
    sid = lax.axis_index("s")

    cols = [_STRIDE * (sid * _CPW + j) for j in range(_CPW)]
    copies = [
        pltpu.async_copy(
            table_t_hbm.at[
                :, pl.ds(pl.multiple_of(cols[j] & ~(_LANES - 1), _LANES), _LANES)
            ],
            blk_v.at[j],
            sem,
        )
        for j in range(_CPW)
    ]
    for c in copies:
        c.wait()

    seq = lax.iota(jnp.int32, _L)
    zeros = jnp.zeros((_L,), jnp.int32)
    for j in range(_CPW):
        lane = zeros + (cols[j] & (_LANES - 1))
        blk = zeros + j
        for q in range(_D // _L):
            vals = plsc.load_gather(blk_v, [blk, seq + q * _L, lane])
            out_v[j, pl.ds(q * _L, _L)] = vals
    pltpu.sync_copy(out_v, out_hbm.at[pl.ds(sid * _CPW, _CPW)])


def kernel(x):
    return _gather_rows(x.T)
